# baseline (speedup 1.0000x reference)
import functools

import jax
import jax.numpy as jnp
from jax import lax
from jax.experimental import pallas as pl
from jax.experimental.pallas import tpu as pltpu
from jax.experimental.pallas import tpu_sc as plsc

_NEG = -1e30
_HS = (0, 1, 62, 63)
_POS = {0: 0, 1: 1, 62: 2, 63: 3}
_EXCL = {0: (62, 63), 1: (0, 63), 2: (0, 1)}


def _border_mask(H, W):
    pos = jnp.arange(H * W)
    rm = (pos[:, None] // W == jnp.arange(H)[None, :]).astype(jnp.float32)
    cm = (pos[:, None] % W == jnp.arange(W)[None, :]).astype(jnp.float32)
    hs = jnp.asarray(_HS)
    tgt = (hs[:, None] * W + hs[None, :]).reshape(1, 16)
    em = (pos[:, None] == tgt).astype(jnp.float32)
    z = jnp.zeros((H * W, 256 - H - W - 16), jnp.float32)
    return jnp.concatenate([rm, cm, em, z], axis=1)


def _gating_body(x_ref, mask_ref, w3_ref, bias_ref, out_ref, tvt_ref,
                 tidx_ref):
    b = pl.program_id(0)
    xb = x_ref[0].astype(jnp.bfloat16).astype(jnp.float32)
    g = jnp.dot(xb, mask_ref[...],
                preferred_element_type=jnp.float32,
                precision=lax.Precision.HIGHEST)
    tot = jnp.sum(g[:, 0:64], axis=1, keepdims=True)

    def rcol(h):
        return g[:, h:h + 1]

    def ccol(w):
        return g[:, 64 + w:65 + w]

    def ecol(i, j):
        k = 128 + 4 * i + j
        return g[:, k:k + 1]

    acc = jnp.zeros((1, 128), jnp.float32)
    for kh in range(3):
        r0, r1 = _EXCL[kh]
        p0, p1 = _POS[r0], _POS[r1]
        rsum = rcol(r0) + rcol(r1)
        for kw in range(3):
            c0, c1 = _EXCL[kw]
            q0, q1 = _POS[c0], _POS[c1]
            csum = ccol(c0) + ccol(c1)
            xsum = ecol(p0, q0) + ecol(p0, q1) + ecol(p1, q0) + ecol(p1, q1)
            s = tot - rsum - csum + xsum
            k = kh * 3 + kw
            acc = acc + jnp.sum(s * w3_ref[k], axis=0, keepdims=True)
    row = acc + bias_ref[...]
    out_ref[pl.ds(b, 1), :] = row
    m = jnp.max(row)
    p = jnp.exp(row - m)
    tv = 1.0 / jnp.sum(p)
    iota = lax.broadcasted_iota(jnp.int32, (1, 128), 1)
    t = jnp.min(jnp.where(row == m, iota, 128))
    tvt_ref[pl.ds(b, 1), :] = jnp.full((1, 128), tv, jnp.float32)
    tidx_ref[pl.ds(b, 1), :] = jnp.full((1, 128), t, jnp.int32)


def _gating_logits(x2, mask, w3, bias2):
    B, C, HW = x2.shape
    return pl.pallas_call(
        _gating_body,
        grid=(B,),
        in_specs=[
            pl.BlockSpec((1, C, HW), lambda b: (b, 0, 0)),
            pl.BlockSpec((HW, 256), lambda b: (0, 0)),
            pl.BlockSpec((9, C, 128), lambda b: (0, 0, 0)),
            pl.BlockSpec((1, 128), lambda b: (0, 0)),
        ],
        out_specs=[
            pl.BlockSpec((B, 128), lambda b: (0, 0)),
            pl.BlockSpec((B, 128), lambda b: (0, 0)),
            pl.BlockSpec((B, 128), lambda b: (0, 0)),
        ],
        out_shape=[
            jax.ShapeDtypeStruct((B, 128), jnp.float32),
            jax.ShapeDtypeStruct((B, 128), jnp.float32),
            jax.ShapeDtypeStruct((B, 128), jnp.int32),
        ],
    )(x2, mask, w3, bias2)


def _shuffle(v, iota, sh):
    return v.at[iota ^ sh].get(mode="promise_in_bounds")


def _sc_route(logits, B):
    info = plsc.get_sparse_core_info()
    mesh = plsc.VectorSubcoreMesh(core_axis_name="c", subcore_axis_name="s")

    @functools.partial(
        pl.kernel,
        mesh=mesh,
        out_type=jax.ShapeDtypeStruct((B, 16), jnp.float32),
        scratch_types=[
            pltpu.VMEM((16,), jnp.float32),
            pltpu.VMEM((16,), jnp.float32),
        ],
    )
    def run(logits_hbm, ew_out, lbuf, ebuf):
        cid = lax.axis_index("c")
        sid = lax.axis_index("s")
        wid = sid * info.num_cores + cid
        b = wid

        @pl.when(wid < B)
        def _():
            pltpu.sync_copy(logits_hbm.at[b, pl.ds(0, 16)], lbuf)
            l = lbuf[...]
            iota = lax.iota(jnp.int32, 16)
            m = l
            for sh in (1, 2, 4, 8):
                m = jnp.maximum(m, _shuffle(m, iota, sh))
            p = jnp.exp(l - m)
            s = p
            for sh in (1, 2, 4, 8):
                s = s + _shuffle(s, iota, sh)
            tv = 1.0 / s
            t = jnp.where(l == m, iota, 16)
            for sh in (1, 2, 4, 8):
                t = jnp.minimum(t, _shuffle(t, iota, sh))
            ebuf[...] = jnp.where(iota == t, tv, 0.0)
            pltpu.sync_copy(ebuf, ew_out.at[b])

    return run(logits)


def _apply_body(we_ref, x_ref, be_ref, tvt_ref, tidx_ref, out_ref):
    b = pl.program_id(0)
    tv = jnp.max(tvt_ref[pl.ds(b, 1), :])
    t = jnp.max(tidx_ref[pl.ds(b, 1), :])
    m = we_ref[t]
    bias = be_ref[t]
    acc = jnp.dot(m, x_ref[0], preferred_element_type=jnp.float32)
    out_ref[0] = tv * (acc + bias)


def _apply(We, x2, be3, tvt, tidx):
    B, C, HW = x2.shape
    E = We.shape[0]
    return pl.pallas_call(
        _apply_body,
        grid=(B,),
        in_specs=[
            pl.BlockSpec((E, C, C), lambda b: (0, 0, 0)),
            pl.BlockSpec((1, C, HW), lambda b: (b, 0, 0)),
            pl.BlockSpec((E, C, 1), lambda b: (0, 0, 0)),
            pl.BlockSpec((B, 128), lambda b: (0, 0)),
            pl.BlockSpec((B, 128), lambda b: (0, 0)),
        ],
        out_specs=pl.BlockSpec((1, C, HW), lambda b: (b, 0, 0)),
        out_shape=jax.ShapeDtypeStruct((B, C, HW), jnp.float32),
    )(We, x2, be3, tvt, tidx)


def kernel(x, Wg, bg, We, be):
    B, C, H, W = x.shape
    E = Wg.shape[0]
    npos = float((H - 2) * (W - 2))

    wg_r = Wg.astype(jnp.bfloat16).astype(jnp.float32)
    w3 = jnp.pad(jnp.transpose(wg_r, (2, 3, 1, 0)).reshape(9, C, E),
                 ((0, 0), (0, 0), (0, 128 - E)))
    bias2 = jnp.concatenate(
        [bg * npos, jnp.full((128 - E,), _NEG, jnp.float32)]).reshape(1, 128)
    mask = _border_mask(H, W)

    x2 = x.reshape(B, C, H * W)
    logits, tvt, tidx = _gating_logits(x2, mask, w3, bias2)
    ew16 = _sc_route(logits, B)
    out2 = _apply(We, x2, be.reshape(E, C, 1), tvt, tidx)
    return out2.reshape(B, C, H, W), ew16[:, :E]


# speedup vs baseline: 1.1929x; 1.1929x over previous
import functools

import jax
import jax.numpy as jnp
from jax import lax
from jax.experimental import pallas as pl
from jax.experimental.pallas import tpu as pltpu
from jax.experimental.pallas import tpu_sc as plsc

_NEG = -1e30
_HS = (0, 1, 62, 63)
_POS = {0: 0, 1: 1, 62: 2, 63: 3}
_EXCL = {0: (62, 63), 1: (0, 63), 2: (0, 1)}


def _border_mask(H, W):
    pos = jnp.arange(H * W)
    rm = (pos[:, None] // W == jnp.arange(H)[None, :]).astype(jnp.float32)
    cm = (pos[:, None] % W == jnp.arange(W)[None, :]).astype(jnp.float32)
    hs = jnp.asarray(_HS)
    tgt = (hs[:, None] * W + hs[None, :]).reshape(1, 16)
    em = (pos[:, None] == tgt).astype(jnp.float32)
    z = jnp.zeros((H * W, 256 - H - W - 16), jnp.float32)
    return jnp.concatenate([rm, cm, em, z], axis=1)


def _gating_body(x_ref, mask_ref, w3_ref, bias_ref, out_ref, tvt_ref,
                 tidx_ref):
    b = pl.program_id(0)
    xb = x_ref[0].astype(jnp.bfloat16)
    g = jnp.dot(xb, mask_ref[...],
                preferred_element_type=jnp.float32)
    tot = jnp.sum(g[:, 0:64], axis=1, keepdims=True)

    def rcol(h):
        return g[:, h:h + 1]

    def ccol(w):
        return g[:, 64 + w:65 + w]

    def ecol(i, j):
        k = 128 + 4 * i + j
        return g[:, k:k + 1]

    acc = jnp.zeros((1, 128), jnp.float32)
    for kh in range(3):
        r0, r1 = _EXCL[kh]
        p0, p1 = _POS[r0], _POS[r1]
        rsum = rcol(r0) + rcol(r1)
        for kw in range(3):
            c0, c1 = _EXCL[kw]
            q0, q1 = _POS[c0], _POS[c1]
            csum = ccol(c0) + ccol(c1)
            xsum = ecol(p0, q0) + ecol(p0, q1) + ecol(p1, q0) + ecol(p1, q1)
            s = tot - rsum - csum + xsum
            k = kh * 3 + kw
            acc = acc + jnp.sum(s * w3_ref[k], axis=0, keepdims=True)
    row = acc + bias_ref[...]
    out_ref[pl.ds(b, 1), :] = row
    m = jnp.max(row)
    p = jnp.exp(row - m)
    tv = 1.0 / jnp.sum(p)
    iota = lax.broadcasted_iota(jnp.int32, (1, 128), 1)
    t = jnp.min(jnp.where(row == m, iota, 128))
    tvt_ref[pl.ds(b, 1), :] = jnp.full((1, 128), tv, jnp.float32)
    tidx_ref[pl.ds(b, 1), :] = jnp.full((1, 128), t, jnp.int32)


def _gating_logits(x2, mask, w3, bias2):
    B, C, HW = x2.shape
    return pl.pallas_call(
        _gating_body,
        grid=(B,),
        in_specs=[
            pl.BlockSpec((1, C, HW), lambda b: (b, 0, 0)),
            pl.BlockSpec((HW, 256), lambda b: (0, 0)),
            pl.BlockSpec((9, C, 128), lambda b: (0, 0, 0)),
            pl.BlockSpec((1, 128), lambda b: (0, 0)),
        ],
        out_specs=[
            pl.BlockSpec((B, 128), lambda b: (0, 0)),
            pl.BlockSpec((B, 128), lambda b: (0, 0)),
            pl.BlockSpec((B, 128), lambda b: (0, 0)),
        ],
        out_shape=[
            jax.ShapeDtypeStruct((B, 128), jnp.float32),
            jax.ShapeDtypeStruct((B, 128), jnp.float32),
            jax.ShapeDtypeStruct((B, 128), jnp.int32),
        ],
    )(x2, mask, w3, bias2)


def _shuffle(v, iota, sh):
    return v.at[iota ^ sh].get(mode="promise_in_bounds")


def _sc_route(logits, B):
    info = plsc.get_sparse_core_info()
    mesh = plsc.VectorSubcoreMesh(core_axis_name="c", subcore_axis_name="s")

    @functools.partial(
        pl.kernel,
        mesh=mesh,
        out_type=jax.ShapeDtypeStruct((B, 16), jnp.float32),
        scratch_types=[
            pltpu.VMEM((16,), jnp.float32),
            pltpu.VMEM((16,), jnp.float32),
        ],
    )
    def run(logits_hbm, ew_out, lbuf, ebuf):
        cid = lax.axis_index("c")
        sid = lax.axis_index("s")
        wid = sid * info.num_cores + cid
        b = wid

        @pl.when(wid < B)
        def _():
            pltpu.sync_copy(logits_hbm.at[b, pl.ds(0, 16)], lbuf)
            l = lbuf[...]
            iota = lax.iota(jnp.int32, 16)
            m = l
            for sh in (1, 2, 4, 8):
                m = jnp.maximum(m, _shuffle(m, iota, sh))
            p = jnp.exp(l - m)
            s = p
            for sh in (1, 2, 4, 8):
                s = s + _shuffle(s, iota, sh)
            tv = 1.0 / s
            t = jnp.where(l == m, iota, 16)
            for sh in (1, 2, 4, 8):
                t = jnp.minimum(t, _shuffle(t, iota, sh))
            ebuf[...] = jnp.where(iota == t, tv, 0.0)
            pltpu.sync_copy(ebuf, ew_out.at[b])

    return run(logits)


def _apply_body(we_ref, x_ref, be_ref, tvt_ref, tidx_ref, out_ref):
    b = pl.program_id(0)
    tv = jnp.max(tvt_ref[pl.ds(b, 1), :])
    t = jnp.max(tidx_ref[pl.ds(b, 1), :])
    m = we_ref[t]
    bias = be_ref[t]
    acc = jnp.dot(m, x_ref[0], preferred_element_type=jnp.float32)
    out_ref[0] = tv * (acc + bias)


def _apply(We, x2, be3, tvt, tidx):
    B, C, HW = x2.shape
    E = We.shape[0]
    return pl.pallas_call(
        _apply_body,
        grid=(B,),
        in_specs=[
            pl.BlockSpec((E, C, C), lambda b: (0, 0, 0)),
            pl.BlockSpec((1, C, HW), lambda b: (b, 0, 0)),
            pl.BlockSpec((E, C, 1), lambda b: (0, 0, 0)),
            pl.BlockSpec((B, 128), lambda b: (0, 0)),
            pl.BlockSpec((B, 128), lambda b: (0, 0)),
        ],
        out_specs=pl.BlockSpec((1, C, HW), lambda b: (b, 0, 0)),
        out_shape=jax.ShapeDtypeStruct((B, C, HW), jnp.float32),
    )(We, x2, be3, tvt, tidx)


def kernel(x, Wg, bg, We, be):
    B, C, H, W = x.shape
    E = Wg.shape[0]
    npos = float((H - 2) * (W - 2))

    wg_r = Wg.astype(jnp.bfloat16).astype(jnp.float32)
    w3 = jnp.pad(jnp.transpose(wg_r, (2, 3, 1, 0)).reshape(9, C, E),
                 ((0, 0), (0, 0), (0, 128 - E)))
    bias2 = jnp.concatenate(
        [bg * npos, jnp.full((128 - E,), _NEG, jnp.float32)]).reshape(1, 128)
    mask = _border_mask(H, W).astype(jnp.bfloat16)

    x2 = x.reshape(B, C, H * W)
    logits, tvt, tidx = _gating_logits(x2, mask, w3, bias2)
    ew16 = _sc_route(logits, B)
    out2 = _apply(We, x2, be.reshape(E, C, 1), tvt, tidx)
    return out2.reshape(B, C, H, W), ew16[:, :E]


# speedup vs baseline: 2.0586x; 1.7258x over previous
import functools

import jax
import jax.numpy as jnp
from jax import lax
from jax.experimental import pallas as pl
from jax.experimental.pallas import tpu as pltpu
from jax.experimental.pallas import tpu_sc as plsc

_NEG = -1e30
_HS = (0, 1, 62, 63)
_POS = {0: 0, 1: 1, 62: 2, 63: 3}
_EXCL = {0: (62, 63), 1: (0, 63), 2: (0, 1)}


def _border_mask(H, W):
    pos = jnp.arange(H * W)
    rm = (pos[None, :] // W == jnp.arange(H)[:, None]).astype(jnp.float32)
    cm = (pos[None, :] % W == jnp.arange(W)[:, None]).astype(jnp.float32)
    hs = jnp.asarray(_HS)
    tgt = (hs[:, None] * W + hs[None, :]).reshape(16, 1)
    em = (pos[None, :] == tgt).astype(jnp.float32)
    z = jnp.zeros((256 - H - W - 16, H * W), jnp.float32)
    return jnp.concatenate([rm, cm, em, z], axis=0)


def _gating_body(x_ref, mask_ref, w3_ref, bias_ref, out_ref, tvt_ref,
                 tidx_ref):
    b = pl.program_id(0)
    xb = x_ref[0].astype(jnp.bfloat16)
    g = jnp.dot(mask_ref[...], xb,
                preferred_element_type=jnp.float32)
    tot = jnp.sum(g[0:64, :], axis=0, keepdims=True)

    def rrow(h):
        return g[h:h + 1, :]

    def crow(w):
        return g[64 + w:65 + w, :]

    def erow(i, j):
        k = 128 + 4 * i + j
        return g[k:k + 1, :]

    acc = jnp.zeros((1, 128), jnp.float32)
    for kh in range(3):
        r0, r1 = _EXCL[kh]
        p0, p1 = _POS[r0], _POS[r1]
        rsum = rrow(r0) + rrow(r1)
        for kw in range(3):
            c0, c1 = _EXCL[kw]
            q0, q1 = _POS[c0], _POS[c1]
            csum = crow(c0) + crow(c1)
            xsum = (erow(p0, q0) + erow(p0, q1)
                    + erow(p1, q0) + erow(p1, q1))
            s = tot - rsum - csum + xsum
            k = kh * 3 + kw
            acc = acc + jnp.dot(s, w3_ref[k],
                                preferred_element_type=jnp.float32)
    row = acc + bias_ref[...]
    out_ref[pl.ds(b, 1), :] = row
    m = jnp.max(row)
    p = jnp.exp(row - m)
    tv = 1.0 / jnp.sum(p)
    iota = lax.broadcasted_iota(jnp.int32, (1, 128), 1)
    t = jnp.min(jnp.where(row == m, iota, 128))
    tvt_ref[pl.ds(b, 1), :] = jnp.full((1, 128), tv, jnp.float32)
    tidx_ref[pl.ds(b, 1), :] = jnp.full((1, 128), t, jnp.int32)


def _gating_logits(xt, mask, w3, bias2):
    B, HW, C = xt.shape
    return pl.pallas_call(
        _gating_body,
        grid=(B,),
        in_specs=[
            pl.BlockSpec((1, HW, C), lambda b: (b, 0, 0)),
            pl.BlockSpec((256, HW), lambda b: (0, 0)),
            pl.BlockSpec((9, C, 128), lambda b: (0, 0, 0)),
            pl.BlockSpec((1, 128), lambda b: (0, 0)),
        ],
        out_specs=[
            pl.BlockSpec((B, 128), lambda b: (0, 0)),
            pl.BlockSpec((B, 128), lambda b: (0, 0)),
            pl.BlockSpec((B, 128), lambda b: (0, 0)),
        ],
        out_shape=[
            jax.ShapeDtypeStruct((B, 128), jnp.float32),
            jax.ShapeDtypeStruct((B, 128), jnp.float32),
            jax.ShapeDtypeStruct((B, 128), jnp.int32),
        ],
    )(xt, mask, w3, bias2)


def _shuffle(v, iota, sh):
    return v.at[iota ^ sh].get(mode="promise_in_bounds")


def _sc_route(logits, B):
    info = plsc.get_sparse_core_info()
    mesh = plsc.VectorSubcoreMesh(core_axis_name="c", subcore_axis_name="s")

    @functools.partial(
        pl.kernel,
        mesh=mesh,
        out_type=jax.ShapeDtypeStruct((B, 16), jnp.float32),
        scratch_types=[
            pltpu.VMEM((16,), jnp.float32),
            pltpu.VMEM((16,), jnp.float32),
        ],
    )
    def run(logits_hbm, ew_out, lbuf, ebuf):
        cid = lax.axis_index("c")
        sid = lax.axis_index("s")
        wid = sid * info.num_cores + cid
        b = wid

        @pl.when(wid < B)
        def _():
            pltpu.sync_copy(logits_hbm.at[b, pl.ds(0, 16)], lbuf)
            l = lbuf[...]
            iota = lax.iota(jnp.int32, 16)
            m = l
            for sh in (1, 2, 4, 8):
                m = jnp.maximum(m, _shuffle(m, iota, sh))
            p = jnp.exp(l - m)
            s = p
            for sh in (1, 2, 4, 8):
                s = s + _shuffle(s, iota, sh)
            tv = 1.0 / s
            t = jnp.where(l == m, iota, 16)
            for sh in (1, 2, 4, 8):
                t = jnp.minimum(t, _shuffle(t, iota, sh))
            ebuf[...] = jnp.where(iota == t, tv, 0.0)
            pltpu.sync_copy(ebuf, ew_out.at[b])

    return run(logits)


def _apply_body(wet_ref, x_ref, be_ref, tvt_ref, tidx_ref, out_ref):
    b = pl.program_id(0)
    tv = jnp.max(tvt_ref[pl.ds(b, 1), :])
    t = jnp.max(tidx_ref[pl.ds(b, 1), :])
    m = wet_ref[t]
    bias = be_ref[pl.ds(t, 1), :]
    acc = jnp.dot(x_ref[0], m, preferred_element_type=jnp.float32)
    out_ref[0] = tv * (acc + bias)


def _apply(WeT, xt, be, tvt, tidx):
    B, HW, C = xt.shape
    E = WeT.shape[0]
    return pl.pallas_call(
        _apply_body,
        grid=(B,),
        in_specs=[
            pl.BlockSpec((E, C, C), lambda b: (0, 0, 0)),
            pl.BlockSpec((1, HW, C), lambda b: (b, 0, 0)),
            pl.BlockSpec((E, C), lambda b: (0, 0)),
            pl.BlockSpec((B, 128), lambda b: (0, 0)),
            pl.BlockSpec((B, 128), lambda b: (0, 0)),
        ],
        out_specs=pl.BlockSpec((1, HW, C), lambda b: (b, 0, 0)),
        out_shape=jax.ShapeDtypeStruct((B, HW, C), jnp.float32),
    )(WeT, xt, be, tvt, tidx)


def kernel(x, Wg, bg, We, be):
    B, C, H, W = x.shape
    E = Wg.shape[0]
    npos = float((H - 2) * (W - 2))

    wg_r = Wg.astype(jnp.bfloat16).astype(jnp.float32)
    w3 = jnp.pad(jnp.transpose(wg_r, (2, 3, 1, 0)).reshape(9, C, E),
                 ((0, 0), (0, 0), (0, 128 - E)))
    bias2 = jnp.concatenate(
        [bg * npos, jnp.full((128 - E,), _NEG, jnp.float32)]).reshape(1, 128)
    mask = _border_mask(H, W).astype(jnp.bfloat16)
    WeT = jnp.transpose(We, (0, 2, 1))

    xt = jnp.transpose(x, (0, 2, 3, 1)).reshape(B, H * W, C)
    logits, tvt, tidx = _gating_logits(xt, mask, w3, bias2)
    ew16 = _sc_route(logits, B)
    y = _apply(WeT, xt, be, tvt, tidx)
    out = jnp.transpose(y.reshape(B, H, W, C), (0, 3, 1, 2))
    return out, ew16[:, :E]
